# baseline (speedup 1.0000x reference)
import functools

import jax
import jax.numpy as jnp
from jax import lax
from jax.experimental import pallas as pl
from jax.experimental.pallas import tpu as pltpu
from jax.experimental.pallas import tpu_sc as plsc

B, L, D = 16384, 50, 16
N = B * L
NC, NS = 2, 16
NW = NC * NS
N_W = N // NW
C = 128
NCH = N_W // C
G = 8
GC = G * C
NG = NCH // G


@jax.jit
def _sc_gather(idx, table):
    mesh = plsc.VectorSubcoreMesh(core_axis_name="c", subcore_axis_name="s")

    @functools.partial(
        pl.kernel,
        mesh=mesh,
        out_type=jax.ShapeDtypeStruct((NW, N_W, D), jnp.float32),
        scratch_types=[
            pltpu.VMEM((NCH, C), jnp.int32),
            pltpu.VMEM((GC, D), jnp.float32),
            pltpu.SemaphoreType.DMA,
        ],
        compiler_params=pltpu.CompilerParams(use_tc_tiling_on_sc=False),
    )
    def k(idx_hbm, table_hbm, out_hbm, idx_v, rows_v, sem):
        wid = lax.axis_index("s") * NC + lax.axis_index("c")
        pltpu.sync_copy(idx_hbm.at[wid], idx_v)

        @pl.loop(0, NG)
        def _group(g):
            cps = [
                pltpu.async_copy(
                    table_hbm.at[idx_v.at[g * G + j]],
                    rows_v.at[pl.ds(j * C, C), :],
                    sem,
                )
                for j in range(G)
            ]
            for cp in cps:
                cp.wait()
            pltpu.sync_copy(rows_v, out_hbm.at[wid, pl.ds(g * GC, GC), :])

    return k(idx, table)


def kernel(indices, table):
    idx = jnp.asarray(indices, jnp.int32).reshape(NW, NCH, C)
    out = _sc_gather(idx, table)
    return out.reshape(B, L, D)


# speedup vs baseline: 1.0102x; 1.0102x over previous
import functools

import jax
import jax.numpy as jnp
from jax import lax
from jax.experimental import pallas as pl
from jax.experimental.pallas import tpu as pltpu
from jax.experimental.pallas import tpu_sc as plsc

B, L, D = 16384, 50, 16
N = B * L
NC, NS = 2, 16
NW = NC * NS
N_W = N // NW
C = 128
NCH = N_W // C
G = 10
GC = G * C
NG = NCH // G


@jax.jit
def _sc_gather(idx, table):
    mesh = plsc.VectorSubcoreMesh(core_axis_name="c", subcore_axis_name="s")

    @functools.partial(
        pl.kernel,
        mesh=mesh,
        out_type=jax.ShapeDtypeStruct((NW, N_W, D), jnp.float32),
        scratch_types=[
            pltpu.VMEM((NCH, C), jnp.int32),
            pltpu.VMEM((2, GC, D), jnp.float32),
            pltpu.SemaphoreType.DMA,
            pltpu.SemaphoreType.DMA,
            pltpu.SemaphoreType.DMA,
            pltpu.SemaphoreType.DMA,
        ],
        compiler_params=pltpu.CompilerParams(use_tc_tiling_on_sc=False),
    )
    def k(idx_hbm, table_hbm, out_hbm, idx_v, rows_v, g0, g1, s0, s1):
        wid = lax.axis_index("s") * NC + lax.axis_index("c")
        gsem = (g0, g1)
        ssem = (s0, s1)
        pltpu.sync_copy(idx_hbm.at[wid], idx_v)

        def fire(g, b):
            for j in range(G):
                pltpu.async_copy(
                    table_hbm.at[idx_v.at[g * G + j]],
                    rows_v.at[b, pl.ds(j * C, C), :],
                    gsem[b],
                )

        def drain_gather(b):
            pltpu.make_async_copy(
                table_hbm.at[pl.ds(0, GC), :], rows_v.at[b], gsem[b]
            ).wait()

        def store(g, b):
            pltpu.async_copy(
                rows_v.at[b], out_hbm.at[wid, pl.ds(g * GC, GC), :], ssem[b]
            )

        def wait_store(b):
            pltpu.make_async_copy(
                rows_v.at[b], out_hbm.at[wid, pl.ds(0, GC), :], ssem[b]
            ).wait()

        fire(0, 0)
        drain_gather(0)
        fire(1, 1)
        store(0, 0)

        @pl.loop(1, NG - 2, step=2)
        def _pair(g):
            drain_gather(1)
            wait_store(0)
            fire(g + 1, 0)
            store(g, 1)
            drain_gather(0)
            wait_store(1)
            fire(g + 2, 1)
            store(g + 1, 0)

        drain_gather(1)
        store(NG - 1, 1)
        wait_store(0)
        wait_store(1)

    return k(idx, table)


def kernel(indices, table):
    idx = jnp.asarray(indices, jnp.int32).reshape(NW, NCH, C)
    out = _sc_gather(idx, table)
    return out.reshape(B, L, D)
